# Initial kernel scaffold; baseline (speedup 1.0000x reference)
import jax
import jax.numpy as jnp
from jax.experimental import pallas as pl


def kernel(x, edge_index, edge_attr, Wn1, bn1, We1, be1, Wu1, bu1, g1, beta1, Wn2, bn2, We2, be2, Wu2, bu2, g2, beta2):
    raise NotImplementedError("write your pallas kernel here")


# speedup vs baseline: 3.1422x; 3.1422x over previous
import jax
import jax.numpy as jnp
from jax import lax
from jax.experimental import pallas as pl
from jax.experimental.pallas import tpu as pltpu
from jax.experimental.pallas import tpu_sc as plsc

N = 10000
E = 320000
D = 128
D_EA = 128
NC = 2
NS = 16
CH = 128
NW = NC * NS
CHUNKS = -(-E // (CH * NW))
EP = CHUNKS * CH * NW
ACC_ROWS = 10112
ZROWS = ACC_ROWS // NS
WROWS = 624
WTAIL = N - NS * WROWS

import functools


@functools.cache
def _mesh():
    return plsc.VectorSubcoreMesh(core_axis_name="c", subcore_axis_name="s")


def _writeback(acc, out_hbm, c, s):
    r0 = s * WROWS
    pltpu.sync_copy(acc.at[pl.ds(r0, WROWS)],
                    out_hbm.at[pl.ds(c * N + r0, WROWS)])

    @pl.when(s == 0)
    def _():
        t0 = NS * WROWS
        pltpu.sync_copy(acc.at[pl.ds(t0, WTAIL)],
                        out_hbm.at[pl.ds(c * N + t0, WTAIL)])


def _gather_seg_sum_body(x_hbm, src_hbm, dst_hbm, z_hbm, px_hbm,
                         acc, src_v, dst_v, rows_v, sem):
    c = lax.axis_index("c")
    s = lax.axis_index("s")
    wid = s * NC + c

    pltpu.sync_copy(z_hbm, acc.at[pl.ds(s * ZROWS, ZROWS)])
    plsc.subcore_barrier()

    base_chunk = wid * CHUNKS

    @pl.loop(0, CHUNKS)
    def _(i):
        e0 = pl.multiple_of((base_chunk + i) * CH, CH)
        pltpu.sync_copy(src_hbm.at[pl.ds(e0, CH)], src_v)
        pltpu.sync_copy(dst_hbm.at[pl.ds(e0, CH)], dst_v)
        pltpu.async_copy(x_hbm.at[src_v], rows_v, sem).wait()
        pltpu.sync_copy(rows_v, acc.at[dst_v], add=True)

    plsc.subcore_barrier()
    _writeback(acc, px_hbm, c, s)


@functools.cache
def _gather_seg_sum():
    return pl.kernel(
        _gather_seg_sum_body,
        out_type=jax.ShapeDtypeStruct((NC * N, D), jnp.float32),
        mesh=_mesh(),
        scratch_types=[
            pltpu.VMEM_SHARED((ACC_ROWS, D), jnp.float32),
            pltpu.VMEM((CH,), jnp.int32),
            pltpu.VMEM((CH,), jnp.int32),
            pltpu.VMEM((CH, D), jnp.float32),
            pltpu.SemaphoreType.DMA,
        ],
    )


def _ea_seg_sum_body(ea_hbm, dst_hbm, z_hbm, pe_hbm, acc, dst_v, ea_v):
    c = lax.axis_index("c")
    s = lax.axis_index("s")
    wid = s * NC + c

    pltpu.sync_copy(z_hbm, acc.at[pl.ds(s * ZROWS, ZROWS)])
    plsc.subcore_barrier()

    base_chunk = wid * CHUNKS

    @pl.loop(0, CHUNKS)
    def _(i):
        e0 = pl.multiple_of((base_chunk + i) * CH, CH)
        pltpu.sync_copy(dst_hbm.at[pl.ds(e0, CH)], dst_v)
        pltpu.sync_copy(ea_hbm.at[pl.ds(e0, CH)], ea_v)
        pltpu.sync_copy(ea_v, acc.at[dst_v], add=True)

    plsc.subcore_barrier()
    _writeback(acc, pe_hbm, c, s)


@functools.cache
def _ea_seg_sum():
    return pl.kernel(
        _ea_seg_sum_body,
        out_type=jax.ShapeDtypeStruct((NC * N, D_EA), jnp.float32),
        mesh=_mesh(),
        scratch_types=[
            pltpu.VMEM_SHARED((ACC_ROWS, D_EA), jnp.float32),
            pltpu.VMEM((CH,), jnp.int32),
            pltpu.VMEM((CH, D_EA), jnp.float32),
        ],
    )


BLK = 1000


def _dense_body(xin, p0, p1, q0, q1, wn, we, bnbe, wua, wub, bu, g, beta,
                out):
    sx = p0[...] + p1[...]
    se = q0[...] + q1[...]
    cnt = se[:, 16:17]
    rinv = 1.0 / jnp.maximum(cnt, 1.0)
    aggr = (jnp.dot(sx, wn[...], preferred_element_type=jnp.float32)
            + jnp.dot(se[:, :16], we[...], preferred_element_type=jnp.float32)
            + cnt * bnbe[...]) * rinv
    h = (jnp.dot(xin[...], wua[...], preferred_element_type=jnp.float32)
         + jnp.dot(aggr, wub[...], preferred_element_type=jnp.float32)
         + bu[...])
    h = jnp.maximum(h, 0.0)
    mu = jnp.mean(h, axis=-1, keepdims=True)
    var = jnp.mean((h - mu) ** 2, axis=-1, keepdims=True)
    out[...] = (h - mu) * jax.lax.rsqrt(var + 1e-5) * g[...] + beta[...]


def _dense_layer(xin, px, pe, Wn, We, bnbe, Wua, Wub, bu, g, beta):
    nb = N // BLK
    full = lambda i: (0, 0)
    return pl.pallas_call(
        _dense_body,
        grid=(nb,),
        in_specs=[
            pl.BlockSpec((BLK, D), lambda i: (i, 0)),
            pl.BlockSpec((BLK, D), lambda i: (i, 0)),
            pl.BlockSpec((BLK, D), lambda i, _n=nb: (i + _n, 0)),
            pl.BlockSpec((BLK, D_EA), lambda i: (i, 0)),
            pl.BlockSpec((BLK, D_EA), lambda i, _n=nb: (i + _n, 0)),
            pl.BlockSpec((D, D), full),
            pl.BlockSpec((16, D), full),
            pl.BlockSpec((1, D), full),
            pl.BlockSpec((D, D), full),
            pl.BlockSpec((D, D), full),
            pl.BlockSpec((1, D), full),
            pl.BlockSpec((1, D), full),
            pl.BlockSpec((1, D), full),
        ],
        out_specs=pl.BlockSpec((BLK, D), lambda i: (i, 0)),
        out_shape=jax.ShapeDtypeStruct((N, D), jnp.float32),
    )(xin, px, px, pe, pe, Wn, We, bnbe, Wua, Wub, bu, g, beta)


def kernel(x, edge_index, edge_attr, Wn1, bn1, We1, be1, Wu1, bu1, g1, beta1,
           Wn2, bn2, We2, be2, Wu2, bu2, g2, beta2):
    src = edge_index[0]
    dst = edge_index[1]
    pad = EP - E
    src_p = jnp.concatenate([src, jnp.zeros((pad,), jnp.int32)])
    dst_p = jnp.concatenate([dst, jnp.full((pad,), N, jnp.int32)])
    ea = jnp.concatenate(
        [edge_attr,
         jnp.ones((E, 1), jnp.float32),
         jnp.zeros((E, D_EA - 17), jnp.float32)], axis=1)
    ea = jnp.concatenate([ea, jnp.zeros((pad, D_EA), jnp.float32)], axis=0)
    zx = jnp.zeros((ZROWS, D), jnp.float32)
    ze = jnp.zeros((ZROWS, D_EA), jnp.float32)

    pe = _ea_seg_sum()(ea, dst_p, ze)
    px = _gather_seg_sum()(x, src_p, dst_p, zx)
    h1 = _dense_layer(x, px, pe, Wn1, We1, (bn1 + be1).reshape(1, D),
                      Wu1[:D], Wu1[D:], bu1.reshape(1, D),
                      g1.reshape(1, D), beta1.reshape(1, D))
    ph = _gather_seg_sum()(h1, src_p, dst_p, zx)
    out = _dense_layer(h1, ph, pe, Wn2, We2, (bn2 + be2).reshape(1, D),
                       Wu2[:D], Wu2[D:], bu2.reshape(1, D),
                       g2.reshape(1, D), beta2.reshape(1, D))
    return out


# speedup vs baseline: 3.4130x; 1.0862x over previous
import jax
import jax.numpy as jnp
from jax import lax
from jax.experimental import pallas as pl
from jax.experimental.pallas import tpu as pltpu
from jax.experimental.pallas import tpu_sc as plsc

N = 10000
E = 320000
D = 128
D_EA = 128
NC = 2
NS = 16
CH = 128
NW = NC * NS
CHUNKS = -(-E // (CH * NW))
EP = CHUNKS * CH * NW
ACC_ROWS = 10112
ZROWS = ACC_ROWS // NS
WROWS = 624
WTAIL = N - NS * WROWS

import functools


@functools.cache
def _mesh():
    return plsc.VectorSubcoreMesh(core_axis_name="c", subcore_axis_name="s")


def _writeback(acc, out_hbm, c, s):
    r0 = s * WROWS
    pltpu.sync_copy(acc.at[pl.ds(r0, WROWS)],
                    out_hbm.at[pl.ds(c * N + r0, WROWS)])

    @pl.when(s == 0)
    def _():
        t0 = NS * WROWS
        pltpu.sync_copy(acc.at[pl.ds(t0, WTAIL)],
                        out_hbm.at[pl.ds(c * N + t0, WTAIL)])


def _gather_seg_sum_body(x_hbm, sd_hbm, z_hbm, px_hbm,
                         acc, sd0, sd1, rows0, rows1, sem0, sem1):
    c = lax.axis_index("c")
    s = lax.axis_index("s")
    wid = s * NC + c

    pltpu.sync_copy(z_hbm, acc.at[pl.ds(s * ZROWS, ZROWS)])
    plsc.subcore_barrier()

    base = wid * CHUNKS
    bufs = ((sd0, rows0, sem0), (sd1, rows1, sem1))

    pltpu.sync_copy(sd_hbm.at[base], sd0)
    pltpu.async_copy(x_hbm.at[sd0.at[0]], rows0, sem0).wait()
    pltpu.sync_copy(rows0, acc.at[sd0.at[1]], add=True)

    @pl.loop(1, CHUNKS, step=2)
    def _(i):
        pltpu.sync_copy(sd_hbm.at[i + base], sd0)
        pltpu.sync_copy(sd_hbm.at[i + base + 1], sd1)
        d0 = pltpu.async_copy(x_hbm.at[sd0.at[0]], rows0, sem0)
        d1 = pltpu.async_copy(x_hbm.at[sd1.at[0]], rows1, sem1)
        d0.wait()
        pltpu.sync_copy(rows0, acc.at[sd0.at[1]], add=True)
        d1.wait()
        pltpu.sync_copy(rows1, acc.at[sd1.at[1]], add=True)

    plsc.subcore_barrier()
    _writeback(acc, px_hbm, c, s)


@functools.cache
def _gather_seg_sum():
    return pl.kernel(
        _gather_seg_sum_body,
        out_type=jax.ShapeDtypeStruct((NC * N, D), jnp.float32),
        mesh=_mesh(),
        scratch_types=[
            pltpu.VMEM_SHARED((ACC_ROWS, D), jnp.float32),
            pltpu.VMEM((2, CH), jnp.int32),
            pltpu.VMEM((2, CH), jnp.int32),
            pltpu.VMEM((CH, D), jnp.float32),
            pltpu.VMEM((CH, D), jnp.float32),
            pltpu.SemaphoreType.DMA,
            pltpu.SemaphoreType.DMA,
        ],
    )


def _ea_seg_sum_body(ea_hbm, sd_hbm, z_hbm, pe_hbm,
                     acc, sd0, sd1, ea0, ea1, sem0, sem1):
    c = lax.axis_index("c")
    s = lax.axis_index("s")
    wid = s * NC + c

    pltpu.sync_copy(z_hbm, acc.at[pl.ds(s * ZROWS, ZROWS)])
    plsc.subcore_barrier()

    base = wid * CHUNKS

    e00 = pl.multiple_of(base * CH, CH)
    pltpu.sync_copy(sd_hbm.at[base], sd0)
    pltpu.async_copy(ea_hbm.at[pl.ds(e00, CH)], ea0, sem0).wait()
    pltpu.sync_copy(ea0, acc.at[sd0.at[1]], add=True)

    @pl.loop(1, CHUNKS, step=2)
    def _(i):
        e0 = pl.multiple_of((base + i) * CH, CH)
        pltpu.sync_copy(sd_hbm.at[i + base], sd0)
        pltpu.sync_copy(sd_hbm.at[i + base + 1], sd1)
        d0 = pltpu.async_copy(ea_hbm.at[pl.ds(e0, CH)], ea0, sem0)
        d1 = pltpu.async_copy(ea_hbm.at[pl.ds(e0 + CH, CH)], ea1, sem1)
        d0.wait()
        pltpu.sync_copy(ea0, acc.at[sd0.at[1]], add=True)
        d1.wait()
        pltpu.sync_copy(ea1, acc.at[sd1.at[1]], add=True)

    plsc.subcore_barrier()
    _writeback(acc, pe_hbm, c, s)


@functools.cache
def _ea_seg_sum():
    return pl.kernel(
        _ea_seg_sum_body,
        out_type=jax.ShapeDtypeStruct((NC * N, D_EA), jnp.float32),
        mesh=_mesh(),
        scratch_types=[
            pltpu.VMEM_SHARED((ACC_ROWS, D_EA), jnp.float32),
            pltpu.VMEM((2, CH), jnp.int32),
            pltpu.VMEM((2, CH), jnp.int32),
            pltpu.VMEM((CH, D_EA), jnp.float32),
            pltpu.VMEM((CH, D_EA), jnp.float32),
            pltpu.SemaphoreType.DMA,
            pltpu.SemaphoreType.DMA,
        ],
    )


BLK = 1000


def _dense_body(xin, p0, p1, q0, q1, wn, we, bnbe, wua, wub, bu, g, beta,
                out):
    sx = p0[...] + p1[...]
    se = q0[...] + q1[...]
    cnt = se[:, 16:17]
    rinv = 1.0 / jnp.maximum(cnt, 1.0)
    aggr = (jnp.dot(sx, wn[...], preferred_element_type=jnp.float32)
            + jnp.dot(se[:, :16], we[...], preferred_element_type=jnp.float32)
            + cnt * bnbe[...]) * rinv
    h = (jnp.dot(xin[...], wua[...], preferred_element_type=jnp.float32)
         + jnp.dot(aggr, wub[...], preferred_element_type=jnp.float32)
         + bu[...])
    h = jnp.maximum(h, 0.0)
    mu = jnp.mean(h, axis=-1, keepdims=True)
    var = jnp.mean((h - mu) ** 2, axis=-1, keepdims=True)
    out[...] = (h - mu) * jax.lax.rsqrt(var + 1e-5) * g[...] + beta[...]


def _dense_layer(xin, px, pe, Wn, We, bnbe, Wua, Wub, bu, g, beta):
    nb = N // BLK
    full = lambda i: (0, 0)
    return pl.pallas_call(
        _dense_body,
        grid=(nb,),
        in_specs=[
            pl.BlockSpec((BLK, D), lambda i: (i, 0)),
            pl.BlockSpec((BLK, D), lambda i: (i, 0)),
            pl.BlockSpec((BLK, D), lambda i, _n=nb: (i + _n, 0)),
            pl.BlockSpec((BLK, D_EA), lambda i: (i, 0)),
            pl.BlockSpec((BLK, D_EA), lambda i, _n=nb: (i + _n, 0)),
            pl.BlockSpec((D, D), full),
            pl.BlockSpec((16, D), full),
            pl.BlockSpec((1, D), full),
            pl.BlockSpec((D, D), full),
            pl.BlockSpec((D, D), full),
            pl.BlockSpec((1, D), full),
            pl.BlockSpec((1, D), full),
            pl.BlockSpec((1, D), full),
        ],
        out_specs=pl.BlockSpec((BLK, D), lambda i: (i, 0)),
        out_shape=jax.ShapeDtypeStruct((N, D), jnp.float32),
    )(xin, px, px, pe, pe, Wn, We, bnbe, Wua, Wub, bu, g, beta)


def kernel(x, edge_index, edge_attr, Wn1, bn1, We1, be1, Wu1, bu1, g1, beta1,
           Wn2, bn2, We2, be2, Wu2, bu2, g2, beta2):
    src = edge_index[0]
    dst = edge_index[1]
    pad = EP - E
    src_p = jnp.concatenate([src, jnp.zeros((pad,), jnp.int32)])
    dst_p = jnp.concatenate([dst, jnp.full((pad,), N, jnp.int32)])
    sd = jnp.stack([src_p.reshape(-1, CH), dst_p.reshape(-1, CH)], axis=1)
    ea = jnp.concatenate(
        [edge_attr,
         jnp.ones((E, 1), jnp.float32),
         jnp.zeros((E, D_EA - 17), jnp.float32)], axis=1)
    ea = jnp.concatenate([ea, jnp.zeros((pad, D_EA), jnp.float32)], axis=0)
    zx = jnp.zeros((ZROWS, D), jnp.float32)
    ze = jnp.zeros((ZROWS, D_EA), jnp.float32)

    pe = _ea_seg_sum()(ea, sd, ze)
    px = _gather_seg_sum()(x, sd, zx)
    h1 = _dense_layer(x, px, pe, Wn1, We1, (bn1 + be1).reshape(1, D),
                      Wu1[:D], Wu1[D:], bu1.reshape(1, D),
                      g1.reshape(1, D), beta1.reshape(1, D))
    ph = _gather_seg_sum()(h1, sd, zx)
    out = _dense_layer(h1, ph, pe, Wn2, We2, (bn2 + be2).reshape(1, D),
                       Wu2[:D], Wu2[D:], bu2.reshape(1, D),
                       g2.reshape(1, D), beta2.reshape(1, D))
    return out
